# baseline (speedup 1.0000x reference)
import functools

import jax
import jax.numpy as jnp
from jax import lax
from jax.experimental import pallas as pl
from jax.experimental.pallas import tpu as pltpu
from jax.experimental.pallas import tpu_sc as plsc

_NUM_V = 1_000_000
_NUM_T = 4_000_000
_NC = 2
_NS = 16
_NW = _NC * _NS
_TPW = _NUM_T // _NW
_SBT = 1000
_SB = 3 * _SBT
_NSB = _TPW // _SBT

_mesh = plsc.VectorSubcoreMesh(core_axis_name="c", subcore_axis_name="s")


@functools.partial(
    pl.kernel,
    mesh=_mesh,
    out_type=jax.ShapeDtypeStruct((_NUM_T, 3, 3), jnp.float32),
    scratch_types=[
        pltpu.VMEM((3, _SB), jnp.int32),
        pltpu.VMEM((2, _SB, 1, 16), jnp.float32),
        pltpu.SemaphoreType.DMA((3,)),
        pltpu.SemaphoreType.DMA((2,)),
        pltpu.SemaphoreType.DMA((2,)),
    ],
    compiler_params=pltpu.CompilerParams(use_tc_tiling_on_sc=False),
)
def _gather_sc(table_hbm, idx_hbm, out_hbm, idx_v, rows_v,
               isem, gsem, ssem):
    wid = lax.axis_index("s") * _NC + lax.axis_index("c")
    base_t = wid * _TPW

    def start_idx(g, i):
        for k in range(3):
            pltpu.async_copy(
                idx_hbm.at[pl.ds(k * _NUM_T + base_t + g * _SBT, _SBT)],
                idx_v.at[i, pl.ds(k * _SBT, _SBT)], isem.at[i])

    def wait_idx(i):
        for k in range(3):
            pltpu.make_async_copy(
                idx_hbm.at[pl.ds(k * _NUM_T + base_t, _SBT)],
                idx_v.at[i, pl.ds(k * _SBT, _SBT)], isem.at[i]).wait()

    def start_gather(i, p):
        pltpu.async_copy(
            table_hbm.at[idx_v.at[i]], rows_v.at[p], gsem.at[p])

    def wait_gather(i, p):
        pltpu.make_async_copy(
            table_hbm.at[idx_v.at[i]], rows_v.at[p], gsem.at[p]).wait()

    def start_store(g, p):
        for k in range(3):
            pltpu.async_copy(
                rows_v.at[p, pl.ds(k * _SBT, _SBT), :, pl.ds(0, 3)],
                out_hbm.at[pl.ds(base_t + g * _SBT, _SBT),
                           pl.ds(k, 1), pl.ds(0, 3)],
                ssem.at[p])

    def wait_store(p):
        for k in range(3):
            pltpu.make_async_copy(
                rows_v.at[p, pl.ds(k * _SBT, _SBT), :, pl.ds(0, 3)],
                out_hbm.at[pl.ds(base_t, _SBT), pl.ds(k, 1), pl.ds(0, 3)],
                ssem.at[p]).wait()

    start_idx(0, 0)

    def body(g, carry):
        p2 = lax.rem(g, 2)
        q2 = 1 - p2
        p3 = lax.rem(g, 3)
        n3 = lax.rem(g + 1, 3)
        m3 = lax.rem(g + 2, 3)

        @pl.when(g + 1 < _NSB)
        def _():
            start_idx(g + 1, n3)

        @pl.when(g >= 2)
        def _():
            wait_store(p2)

        wait_idx(p3)
        start_gather(p3, p2)

        @pl.when(g >= 1)
        def _():
            wait_gather(m3, q2)
            start_store(g - 1, q2)

        return carry

    lax.fori_loop(0, _NSB, body, 0)

    p_last = (_NSB - 1) % 2
    wait_gather((_NSB - 1) % 3, p_last)
    start_store(_NSB - 1, p_last)
    wait_store(1 - p_last)
    wait_store(p_last)


def kernel(vertices, triangles):
    table16 = jnp.pad(vertices[:, None, :], ((0, 0), (0, 0), (0, 13)))
    idx_km = triangles.T.reshape(3 * _NUM_T)
    return _gather_sc(table16, idx_km)


# speedup vs baseline: 1.0144x; 1.0144x over previous
import functools

import jax
import jax.numpy as jnp
from jax import lax
from jax.experimental import pallas as pl
from jax.experimental.pallas import tpu as pltpu
from jax.experimental.pallas import tpu_sc as plsc

_NUM_V = 1_000_000
_NUM_T = 4_000_000
_NC = 2
_NS = 16
_NW = _NC * _NS
_TPW = _NUM_T // _NW
_SBT = 200
_SB = 3 * _SBT
_NSB = _TPW // _SBT
_ND = 8
_NI = 10

_mesh = plsc.VectorSubcoreMesh(core_axis_name="c", subcore_axis_name="s")


@functools.partial(
    pl.kernel,
    mesh=_mesh,
    out_type=jax.ShapeDtypeStruct((_NUM_T, 3, 3), jnp.float32),
    scratch_types=[
        pltpu.VMEM((_NI, _SB), jnp.int32),
        pltpu.VMEM((_ND, _SB, 1, 16), jnp.float32),
        pltpu.SemaphoreType.DMA((_NI,)),
        pltpu.SemaphoreType.DMA((_ND,)),
        pltpu.SemaphoreType.DMA((_ND,)),
    ],
    compiler_params=pltpu.CompilerParams(use_tc_tiling_on_sc=False),
)
def _gather_sc(table_hbm, idx_hbm, out_hbm, idx_v, rows_v,
               isem, gsem, ssem):
    wid = lax.axis_index("s") * _NC + lax.axis_index("c")
    base_t = wid * _TPW

    def start_idx(g, i):
        for k in range(3):
            pltpu.async_copy(
                idx_hbm.at[pl.ds(k * _NUM_T + base_t + g * _SBT, _SBT)],
                idx_v.at[i, pl.ds(k * _SBT, _SBT)], isem.at[i])

    def wait_idx(i):
        for k in range(3):
            pltpu.make_async_copy(
                idx_hbm.at[pl.ds(k * _NUM_T + base_t, _SBT)],
                idx_v.at[i, pl.ds(k * _SBT, _SBT)], isem.at[i]).wait()

    def start_gather(i, p):
        pltpu.async_copy(
            table_hbm.at[idx_v.at[i]], rows_v.at[p], gsem.at[p])

    def wait_gather(i, p):
        pltpu.make_async_copy(
            table_hbm.at[idx_v.at[i]], rows_v.at[p], gsem.at[p]).wait()

    def start_store(g, p):
        for k in range(3):
            pltpu.async_copy(
                rows_v.at[p, pl.ds(k * _SBT, _SBT), :, pl.ds(0, 3)],
                out_hbm.at[pl.ds(base_t + g * _SBT, _SBT),
                           pl.ds(k, 1), pl.ds(0, 3)],
                ssem.at[p])

    def wait_store(p):
        for k in range(3):
            pltpu.make_async_copy(
                rows_v.at[p, pl.ds(k * _SBT, _SBT), :, pl.ds(0, 3)],
                out_hbm.at[pl.ds(base_t, _SBT), pl.ds(k, 1), pl.ds(0, 3)],
                ssem.at[p]).wait()

    for g0 in range(2):
        start_idx(g0, g0)

    def body(g, carry):
        pd = lax.rem(g, _ND)
        qd = lax.rem(g + 1, _ND)
        pi = lax.rem(g, _NI)
        ni = lax.rem(g + 2, _NI)
        mi = lax.rem(g + (_NI - (_ND - 1)), _NI)

        @pl.when(g + 2 < _NSB)
        def _():
            start_idx(g + 2, ni)

        @pl.when(g >= _ND)
        def _():
            wait_store(pd)

        wait_idx(pi)
        start_gather(pi, pd)

        @pl.when(g >= _ND - 1)
        def _():
            wait_gather(mi, qd)
            start_store(g - (_ND - 1), qd)

        return carry

    lax.fori_loop(0, _NSB, body, 0)

    for r in range(_ND - 1):
        g = _NSB - (_ND - 1) + r
        pd = g % _ND
        wait_gather(g % _NI, pd)
        start_store(g, pd)
    for r in range(_ND):
        g = _NSB - _ND + r
        wait_store(g % _ND)


def kernel(vertices, triangles):
    table16 = jnp.pad(vertices, ((0, 0), (0, 13)))
    idx_km = triangles.T.reshape(3 * _NUM_T)
    return _gather_sc(table16.reshape(_NUM_V, 1, 16), idx_km)


# speedup vs baseline: 1.0195x; 1.0051x over previous
import functools

import jax
import jax.numpy as jnp
from jax import lax
from jax.experimental import pallas as pl
from jax.experimental.pallas import tpu as pltpu
from jax.experimental.pallas import tpu_sc as plsc

_NUM_V = 1_000_000
_NUM_T = 4_000_000
_NC = 2
_NS = 16
_NW = _NC * _NS
_TPW = _NUM_T // _NW
_SBT = 1000
_SB = 3 * _SBT
_NSB = _TPW // _SBT
_ND = 2
_NI = 4

_mesh = plsc.VectorSubcoreMesh(core_axis_name="c", subcore_axis_name="s")


@functools.partial(
    pl.kernel,
    mesh=_mesh,
    out_type=jax.ShapeDtypeStruct((_NUM_T, 3, 3), jnp.float32),
    scratch_types=[
        pltpu.VMEM((_NI, _SB), jnp.int32),
        pltpu.VMEM((_ND, _SB, 1, 16), jnp.float32),
        pltpu.SemaphoreType.DMA((_NI,)),
        pltpu.SemaphoreType.DMA((_ND,)),
        pltpu.SemaphoreType.DMA((_ND,)),
    ],
    compiler_params=pltpu.CompilerParams(use_tc_tiling_on_sc=False),
)
def _gather_sc(table_hbm, idx_hbm, out_hbm, idx_v, rows_v,
               isem, gsem, ssem):
    wid = lax.axis_index("s") * _NC + lax.axis_index("c")
    base_t = wid * _TPW

    def start_idx(g, i):
        for k in range(3):
            pltpu.async_copy(
                idx_hbm.at[pl.ds(k * _NUM_T + base_t + g * _SBT, _SBT)],
                idx_v.at[i, pl.ds(k * _SBT, _SBT)], isem.at[i])

    def wait_idx(i):
        for k in range(3):
            pltpu.make_async_copy(
                idx_hbm.at[pl.ds(k * _NUM_T + base_t, _SBT)],
                idx_v.at[i, pl.ds(k * _SBT, _SBT)], isem.at[i]).wait()

    def start_gather(i, p):
        pltpu.async_copy(
            table_hbm.at[idx_v.at[i]], rows_v.at[p], gsem.at[p])

    def wait_gather(i, p):
        pltpu.make_async_copy(
            table_hbm.at[idx_v.at[i]], rows_v.at[p], gsem.at[p]).wait()

    def start_store(g, p):
        for k in range(3):
            pltpu.async_copy(
                rows_v.at[p, pl.ds(k * _SBT, _SBT), :, pl.ds(0, 3)],
                out_hbm.at[pl.ds(base_t + g * _SBT, _SBT),
                           pl.ds(k, 1), pl.ds(0, 3)],
                ssem.at[p])

    def wait_store(p):
        for k in range(3):
            pltpu.make_async_copy(
                rows_v.at[p, pl.ds(k * _SBT, _SBT), :, pl.ds(0, 3)],
                out_hbm.at[pl.ds(base_t, _SBT), pl.ds(k, 1), pl.ds(0, 3)],
                ssem.at[p]).wait()

    for g0 in range(2):
        start_idx(g0, g0)

    def body(g, carry):
        pd = lax.rem(g, _ND)
        qd = lax.rem(g + 1, _ND)
        pi = lax.rem(g, _NI)
        ni = lax.rem(g + 2, _NI)
        mi = lax.rem(g + (_NI - (_ND - 1)), _NI)

        @pl.when(g + 2 < _NSB)
        def _():
            start_idx(g + 2, ni)

        @pl.when(g >= _ND)
        def _():
            wait_store(pd)

        wait_idx(pi)
        start_gather(pi, pd)

        @pl.when(g >= _ND - 1)
        def _():
            wait_gather(mi, qd)
            start_store(g - (_ND - 1), qd)

        return carry

    lax.fori_loop(0, _NSB, body, 0)

    for r in range(_ND - 1):
        g = _NSB - (_ND - 1) + r
        pd = g % _ND
        wait_gather(g % _NI, pd)
        start_store(g, pd)
    for r in range(_ND):
        g = _NSB - _ND + r
        wait_store(g % _ND)


def kernel(vertices, triangles):
    table16 = jnp.pad(vertices, ((0, 0), (0, 13)))
    idx_km = triangles.T.reshape(3 * _NUM_T)
    return _gather_sc(table16.reshape(_NUM_V, 1, 16), idx_km)


# speedup vs baseline: 1.0202x; 1.0007x over previous
import functools

import jax
import jax.numpy as jnp
from jax import lax
from jax.experimental import pallas as pl
from jax.experimental.pallas import tpu as pltpu
from jax.experimental.pallas import tpu_sc as plsc

_NUM_V = 1_000_000
_NUM_T = 4_000_000
_NC = 2
_NS = 16
_NW = _NC * _NS
_TPW = _NUM_T // _NW
_SBT = 1000
_SB = 3 * _SBT
_NSB = _TPW // _SBT
_ND = 2
_NI = 4

_mesh = plsc.VectorSubcoreMesh(
    core_axis_name="c", subcore_axis_name="s", num_cores=_NC)


@functools.partial(
    pl.kernel,
    mesh=_mesh,
    out_type=jax.ShapeDtypeStruct((_NUM_T, 3, 3), jnp.float32),
    scratch_types=[
        pltpu.VMEM((_NI, _SB), jnp.int32),
        pltpu.VMEM((_ND, _SB, 1, 16), jnp.float32),
        pltpu.SemaphoreType.DMA((_NI,)),
        pltpu.SemaphoreType.DMA((_ND,)),
        pltpu.SemaphoreType.DMA((_ND,)),
    ],
    compiler_params=pltpu.CompilerParams(use_tc_tiling_on_sc=False),
)
def _gather_sc(table_hbm, idx_hbm, out_hbm, idx_v, rows_v,
               isem, gsem, ssem):
    wid = lax.axis_index("s") * _NC + lax.axis_index("c")
    base_t = wid * _TPW

    def start_idx(g, i):
        for k in range(3):
            pltpu.async_copy(
                idx_hbm.at[pl.ds(k * _NUM_T + base_t + g * _SBT, _SBT)],
                idx_v.at[i, pl.ds(k * _SBT, _SBT)], isem.at[i])

    def wait_idx(i):
        for k in range(3):
            pltpu.make_async_copy(
                idx_hbm.at[pl.ds(k * _NUM_T + base_t, _SBT)],
                idx_v.at[i, pl.ds(k * _SBT, _SBT)], isem.at[i]).wait()

    def start_gather(i, p):
        pltpu.async_copy(
            table_hbm.at[idx_v.at[i]], rows_v.at[p], gsem.at[p])

    def wait_gather(i, p):
        pltpu.make_async_copy(
            table_hbm.at[idx_v.at[i]], rows_v.at[p], gsem.at[p]).wait()

    def start_store(g, p):
        for k in range(3):
            pltpu.async_copy(
                rows_v.at[p, pl.ds(k * _SBT, _SBT), :, pl.ds(0, 3)],
                out_hbm.at[pl.ds(base_t + g * _SBT, _SBT),
                           pl.ds(k, 1), pl.ds(0, 3)],
                ssem.at[p])

    def wait_store(p):
        for k in range(3):
            pltpu.make_async_copy(
                rows_v.at[p, pl.ds(k * _SBT, _SBT), :, pl.ds(0, 3)],
                out_hbm.at[pl.ds(base_t, _SBT), pl.ds(k, 1), pl.ds(0, 3)],
                ssem.at[p]).wait()

    for g0 in range(2):
        start_idx(g0, g0)

    def body(g, carry):
        pd = lax.rem(g, _ND)
        qd = lax.rem(g + 1, _ND)
        pi = lax.rem(g, _NI)
        ni = lax.rem(g + 2, _NI)
        mi = lax.rem(g + (_NI - (_ND - 1)), _NI)

        @pl.when(g + 2 < _NSB)
        def _():
            start_idx(g + 2, ni)

        @pl.when(g >= _ND)
        def _():
            wait_store(pd)

        wait_idx(pi)
        start_gather(pi, pd)

        @pl.when(g >= _ND - 1)
        def _():
            wait_gather(mi, qd)
            start_store(g - (_ND - 1), qd)

        return carry

    lax.fori_loop(0, _NSB, body, 0)

    for r in range(_ND - 1):
        g = _NSB - (_ND - 1) + r
        pd = g % _ND
        wait_gather(g % _NI, pd)
        start_store(g, pd)
    for r in range(_ND):
        g = _NSB - _ND + r
        wait_store(g % _ND)


def kernel(vertices, triangles):
    table16 = jnp.pad(vertices, ((0, 0), (0, 13)))
    idx_km = triangles.T.reshape(3 * _NUM_T)
    return _gather_sc(table16.reshape(_NUM_V, 1, 16), idx_km)
